# Initial kernel scaffold; baseline (speedup 1.0000x reference)
import jax
import jax.numpy as jnp
from jax.experimental import pallas as pl


def kernel(x, edge_feat, edge_index, Wih_n, Whh_n, bih_n, bhh_n, Wih_e, Whh_e, bih_e, bhh_e, W_nmpn, b_nmpn, W_empn, b_empn, gamma_n, beta_n, gamma_e, beta_e, W_fc, b_fc):
    raise NotImplementedError("write your pallas kernel here")


# speedup vs baseline: 1.9627x; 1.9627x over previous
import functools

import jax
import jax.numpy as jnp
from jax import lax
from jax.experimental import pallas as pl
from jax.experimental.pallas import tpu as pltpu
from jax.experimental.pallas import tpu_sc as plsc

F32 = jnp.float32
NUM_ITER = 13
NEG_SLOPE = 0.01
BN_EPS = 1e-5


def _leaky(z):
    return jnp.where(z > 0, z, NEG_SLOPE * z)


def _lstm_body(with_msg, y, aff_a, aff_b, h, c, wih_t, whh_t, bias, wp_t, bp,
               *rest):
    if with_msg:
        h_out, c_out, post_out = rest
    else:
        (h_out, c_out) = rest
    xin = y[...] * aff_a[...] + aff_b[...]
    gates = (jnp.dot(xin, wih_t[...], preferred_element_type=F32)
             + jnp.dot(h[...], whh_t[...], preferred_element_type=F32)
             + bias[...])
    hh = h.shape[1]
    gi = jax.nn.sigmoid(gates[:, 0:hh])
    gf = jax.nn.sigmoid(gates[:, hh:2 * hh])
    gg = jnp.tanh(gates[:, 2 * hh:3 * hh])
    go = jax.nn.sigmoid(gates[:, 3 * hh:4 * hh])
    c_new = gf * c[...] + gi * gg
    h_new = go * jnp.tanh(c_new)
    h_out[...] = h_new
    c_out[...] = c_new
    if with_msg:
        post_out[...] = jnp.dot(h_new, wp_t[...],
                                preferred_element_type=F32) + bp[...]


def _lstm_call(y, aff_a, aff_b, h, c, wih_t, whh_t, bias, wp_t, bp, tile,
               with_msg):
    n, fin = y.shape
    hh = h.shape[1]
    grid = n // tile
    full = lambda a: pl.BlockSpec(a.shape, lambda i: (0,) * a.ndim)
    in_specs = [
        pl.BlockSpec((tile, fin), lambda i: (i, 0)),
        full(aff_a), full(aff_b),
        pl.BlockSpec((tile, hh), lambda i: (i, 0)),
        pl.BlockSpec((tile, hh), lambda i: (i, 0)),
        full(wih_t), full(whh_t), full(bias), full(wp_t), full(bp),
    ]
    out_specs = [
        pl.BlockSpec((tile, hh), lambda i: (i, 0)),
        pl.BlockSpec((tile, hh), lambda i: (i, 0)),
    ]
    out_shape = [
        jax.ShapeDtypeStruct((n, hh), F32),
        jax.ShapeDtypeStruct((n, hh), F32),
    ]
    if with_msg:
        pw = wp_t.shape[1]
        out_specs.append(pl.BlockSpec((tile, pw), lambda i: (i, 0)))
        out_shape.append(jax.ShapeDtypeStruct((n, pw), F32))
    return pl.pallas_call(
        functools.partial(_lstm_body, with_msg),
        grid=(grid,),
        in_specs=in_specs,
        out_specs=out_specs,
        out_shape=out_shape,
    )(y, aff_a, aff_b, h, c, wih_t, whh_t, bias, wp_t, bp)


def _edge_msg_body(y, aff_a, aff_b, h, c, g1, g3, wih_t, whh_t, bias, w2_t,
                   bmsg, h_out, c_out, ymsg_out, stats_out, acc):
    i = pl.program_id(0)
    xin = y[...] * aff_a[...] + aff_b[...]
    gates = (jnp.dot(xin, wih_t[...], preferred_element_type=F32)
             + jnp.dot(h[...], whh_t[...], preferred_element_type=F32)
             + bias[...])
    hh = h.shape[1]
    gi = jax.nn.sigmoid(gates[:, 0:hh])
    gf = jax.nn.sigmoid(gates[:, hh:2 * hh])
    gg = jnp.tanh(gates[:, 2 * hh:3 * hh])
    go = jax.nn.sigmoid(gates[:, 3 * hh:4 * hh])
    c_new = gf * c[...] + gi * gg
    h_new = go * jnp.tanh(c_new)
    h_out[...] = h_new
    c_out[...] = c_new
    z = (g1[...] + g3[...]
         + jnp.dot(h_new, w2_t[...], preferred_element_type=F32) + bmsg[...])
    ym = _leaky(z)
    ymsg_out[...] = ym

    @pl.when(i == 0)
    def _():
        acc[...] = jnp.zeros_like(acc)

    acc[0:1, :] = acc[0:1, :] + jnp.sum(ym, axis=0, keepdims=True)
    acc[1:2, :] = acc[1:2, :] + jnp.sum(ym * ym, axis=0, keepdims=True)
    stats_out[...] = acc[...]


def _edge_msg_call(y, aff_a, aff_b, h, c, g1, g3, wih_t, whh_t, bias, w2_t,
                   bmsg, tile):
    e, fin = y.shape
    hh = h.shape[1]
    fm = w2_t.shape[1]
    grid = e // tile
    full = lambda a: pl.BlockSpec(a.shape, lambda i: (0,) * a.ndim)
    row = lambda w: pl.BlockSpec((tile, w), lambda i: (i, 0))
    return pl.pallas_call(
        _edge_msg_body,
        grid=(grid,),
        in_specs=[row(fin), full(aff_a), full(aff_b), row(hh), row(hh),
                  row(fm), row(fm), full(wih_t), full(whh_t), full(bias),
                  full(w2_t), full(bmsg)],
        out_specs=[row(hh), row(hh), row(fm),
                   pl.BlockSpec((8, fm), lambda i: (0, 0))],
        out_shape=[jax.ShapeDtypeStruct((e, hh), F32),
                   jax.ShapeDtypeStruct((e, hh), F32),
                   jax.ShapeDtypeStruct((e, fm), F32),
                   jax.ShapeDtypeStruct((8, fm), F32)],
        scratch_shapes=[pltpu.VMEM((8, fm), F32)],
    )(y, aff_a, aff_b, h, c, g1, g3, wih_t, whh_t, bias, w2_t, bmsg)


def _node_msg_body(agg0, agg1, h, wa_t, wh_t, bias, y_out, stats_out, acc):
    i = pl.program_id(0)
    agg = agg0[...] + agg1[...]
    z = (jnp.dot(agg, wa_t[...], preferred_element_type=F32)
         + jnp.dot(h[...], wh_t[...], preferred_element_type=F32) + bias[...])
    ym = _leaky(z)
    y_out[...] = ym

    @pl.when(i == 0)
    def _():
        acc[...] = jnp.zeros_like(acc)

    acc[0:1, :] = acc[0:1, :] + jnp.sum(ym, axis=0, keepdims=True)
    acc[1:2, :] = acc[1:2, :] + jnp.sum(ym * ym, axis=0, keepdims=True)
    stats_out[...] = acc[...]


def _node_msg_call(agg0, agg1, h, wa_t, wh_t, bias, tile):
    n, hh = h.shape
    fo = wa_t.shape[1]
    grid = n // tile
    full = lambda a: pl.BlockSpec(a.shape, lambda i: (0,) * a.ndim)
    row = lambda w: pl.BlockSpec((tile, w), lambda i: (i, 0))
    return pl.pallas_call(
        _node_msg_body,
        grid=(grid,),
        in_specs=[row(hh), row(hh), row(hh), full(wa_t), full(wh_t),
                  full(bias)],
        out_specs=[row(fo), pl.BlockSpec((8, fo), lambda i: (0, 0))],
        out_shape=[jax.ShapeDtypeStruct((n, fo), F32),
                   jax.ShapeDtypeStruct((8, fo), F32)],
        scratch_shapes=[pltpu.VMEM((8, fo), F32)],
    )(agg0, agg1, h, wa_t, wh_t, bias)


_NWORK = 32
_CH = 40
_GRP = 5
_SUP = _CH * _GRP
_NPAD = 10240


def _gather_pair(p1, p3, src3, dst3):
    n, fm = p1.shape
    nch = src3.shape[1]
    e = _NWORK * nch * _CH
    ew = nch * _CH
    nsup = nch // _GRP
    mesh = plsc.VectorSubcoreMesh(core_axis_name="c", subcore_axis_name="s")

    @functools.partial(
        pl.kernel,
        out_type=(jax.ShapeDtypeStruct((e, fm), F32),
                  jax.ShapeDtypeStruct((e, fm), F32)),
        mesh=mesh,
        scratch_types=[
            pltpu.VMEM((nch, _CH), jnp.int32),
            pltpu.VMEM((nch, _CH), jnp.int32),
            pltpu.VMEM((_SUP, fm), F32),
            pltpu.VMEM((_SUP, fm), F32),
            pltpu.SemaphoreType.DMA,
            pltpu.SemaphoreType.DMA,
        ],
        compiler_params=pltpu.CompilerParams(use_tc_tiling_on_sc=False),
    )
    def gk(p1_hbm, p3_hbm, src_hbm, dst_hbm, g1_hbm, g3_hbm, idx_s, idx_d,
           r1, r2, sem1, sem2):
        c = lax.axis_index("c")
        s = lax.axis_index("s")
        wid = s * 2 + c
        pltpu.sync_copy(src_hbm.at[wid], idx_s)
        pltpu.sync_copy(dst_hbm.at[wid], idx_d)

        def body(jj, carry):
            cps = []
            for k in range(_GRP):
                j = jj * _GRP + k
                cps.append(pltpu.async_copy(
                    p1_hbm.at[idx_s.at[j]], r1.at[pl.ds(k * _CH, _CH)], sem1))
                cps.append(pltpu.async_copy(
                    p3_hbm.at[idx_d.at[j]], r2.at[pl.ds(k * _CH, _CH)], sem2))
            for cp in cps:
                cp.wait()
            base = wid * ew + jj * _SUP
            pltpu.sync_copy(r1, g1_hbm.at[pl.ds(base, _SUP)])
            pltpu.sync_copy(r2, g3_hbm.at[pl.ds(base, _SUP)])
            return carry

        lax.fori_loop(0, nsup, body, 0)

    return gk(p1, p3, src3, dst3)


def _segment_partials(h_e, dst3, zeros_nh):
    e, hh = h_e.shape
    npad = zeros_nh.shape[0]
    nch = dst3.shape[1]
    ew = nch * _CH
    nsup = nch // _GRP
    rps = npad // 16
    mesh = plsc.VectorSubcoreMesh(core_axis_name="c", subcore_axis_name="s")

    @functools.partial(
        pl.kernel,
        out_type=jax.ShapeDtypeStruct((2 * npad, hh), F32),
        mesh=mesh,
        scratch_types=[
            pltpu.VMEM((nch, _CH), jnp.int32),
            pltpu.VMEM((_SUP, hh), F32),
            pltpu.VMEM_SHARED((npad, hh), F32),
        ],
        compiler_params=pltpu.CompilerParams(use_tc_tiling_on_sc=False),
    )
    def sk(he_hbm, dst_hbm, zero_hbm, out_hbm, idx_d, buf, shared):
        c = lax.axis_index("c")
        s = lax.axis_index("s")
        wid = s * 2 + c
        pltpu.sync_copy(zero_hbm.at[pl.ds(s * rps, rps)],
                        shared.at[pl.ds(s * rps, rps)])
        pltpu.sync_copy(dst_hbm.at[wid], idx_d)
        plsc.subcore_barrier()

        def body(jj, carry):
            pltpu.sync_copy(he_hbm.at[pl.ds(wid * ew + jj * _SUP, _SUP)], buf)
            for k in range(_GRP):
                pltpu.sync_copy(buf.at[pl.ds(k * _CH, _CH)],
                                shared.at[idx_d.at[jj * _GRP + k]], add=True)
            return carry

        lax.fori_loop(0, nsup, body, 0)
        plsc.subcore_barrier()
        pltpu.sync_copy(shared.at[pl.ds(s * rps, rps)],
                        out_hbm.at[pl.ds(c * npad + s * rps, rps)])

    return sk(h_e, dst3, zeros_nh)


def _bn_affine(stats, gamma, beta, count):
    s = stats[0, :]
    q = stats[1, :]
    m = s / count
    v = jnp.maximum(q / count - m * m, 0.0)
    a = gamma * lax.rsqrt(v + BN_EPS)
    b = beta - m * a
    return a[None, :], b[None, :]


def kernel(x, edge_feat, edge_index, Wih_n, Whh_n, bih_n, bhh_n, Wih_e,
           Whh_e, bih_e, bhh_e, W_nmpn, b_nmpn, W_empn, b_empn, gamma_n,
           beta_n, gamma_e, beta_e, W_fc, b_fc):
    n, nf = x.shape
    e, ef = edge_feat.shape
    hh = Whh_n.shape[1]
    nc = W_fc.shape[0]
    ew = e // _NWORK
    nch = ew // _CH

    wih_n_t = Wih_n.T
    whh_n_t = Whh_n.T
    bias_n = (bih_n + bhh_n)[None, :]
    wih_e_t = Wih_e.T
    whh_e_t = Whh_e.T
    bias_e = (bih_e + bhh_e)[None, :]
    w1_t = W_empn[:, 0:hh].T
    w2_t = W_empn[:, hh:2 * hh].T
    w3_t = W_empn[:, 2 * hh:3 * hh].T
    wp_t = jnp.concatenate([w1_t, w3_t], axis=1)
    bp = jnp.zeros((1, 2 * ef), F32)
    wfc_pad = jnp.zeros((hh, 2 * ef), F32).at[:, 0:nc].set(W_fc.T)
    bfc_pad = jnp.zeros((1, 2 * ef), F32).at[0, 0:nc].set(b_fc)
    wa_t = W_nmpn[:, 0:hh].T
    wh_t = W_nmpn[:, hh:2 * hh].T
    bias_nm = b_nmpn[None, :]
    bmsg = b_empn[None, :]

    src3 = edge_index[0].reshape(_NWORK, nch, _CH)
    dst3 = edge_index[1].reshape(_NWORK, nch, _CH)
    zeros_nh = jnp.zeros((_NPAD, hh), F32)

    h_n = jnp.zeros((n, hh), F32)
    c_n = jnp.zeros((n, hh), F32)
    h_e = jnp.zeros((e, hh), F32)
    c_e = jnp.zeros((e, hh), F32)
    y_n = x
    y_e = edge_feat
    an = jnp.ones((1, nf), F32)
    bn = jnp.zeros((1, nf), F32)
    ae = jnp.ones((1, ef), F32)
    be = jnp.zeros((1, ef), F32)

    for it in range(NUM_ITER):
        last = it == NUM_ITER - 1
        wpost = wfc_pad if last else wp_t
        bpost = bfc_pad if last else bp
        h_n, c_n, post = _lstm_call(y_n, an, bn, h_n, c_n, wih_n_t, whh_n_t,
                                    bias_n, wpost, bpost, tile=2000,
                                    with_msg=True)
        if last:
            return post[:, 0:nc]
        if it <= NUM_ITER - 3:
            p1 = post[:, 0:ef]
            p3 = post[:, ef:2 * ef]
            g1, g3 = _gather_pair(p1, p3, src3, dst3)
            h_e, c_e, y_e, stats_e = _edge_msg_call(
                y_e, ae, be, h_e, c_e, g1, g3, wih_e_t, whh_e_t, bias_e,
                w2_t, bmsg, tile=5000)
            ae, be = _bn_affine(stats_e, gamma_e, beta_e, float(e))
        else:
            h_e, c_e = _lstm_call(y_e, ae, be, h_e, c_e, wih_e_t, whh_e_t,
                                  bias_e, w2_t, bmsg, tile=5000,
                                  with_msg=False)
        agg2 = _segment_partials(h_e, dst3, zeros_nh)
        y_n, stats_n = _node_msg_call(agg2[0:n], agg2[_NPAD:_NPAD + n], h_n,
                                      wa_t, wh_t, bias_nm, tile=2000)
        an, bn = _bn_affine(stats_n, gamma_n, beta_n, float(n))


# speedup vs baseline: 2.1520x; 1.0964x over previous
import functools

import jax
import jax.numpy as jnp
from jax import lax
from jax.experimental import pallas as pl
from jax.experimental.pallas import tpu as pltpu
from jax.experimental.pallas import tpu_sc as plsc

F32 = jnp.float32
NUM_ITER = 13
NEG_SLOPE = 0.01
BN_EPS = 1e-5


def _leaky(z):
    return jnp.where(z > 0, z, NEG_SLOPE * z)


def _lstm_body(nproj, y, aff_a, aff_b, h, c, wih_t, whh_t, bias, wp1_t, bp1,
               wp2_t, bp2, *rest):
    h_out, c_out = rest[0], rest[1]
    xin = y[...] * aff_a[...] + aff_b[...]
    gates = (jnp.dot(xin, wih_t[...], preferred_element_type=F32)
             + jnp.dot(h[...], whh_t[...], preferred_element_type=F32)
             + bias[...])
    hh = h.shape[1]
    gi = jax.nn.sigmoid(gates[:, 0:hh])
    gf = jax.nn.sigmoid(gates[:, hh:2 * hh])
    gg = jnp.tanh(gates[:, 2 * hh:3 * hh])
    go = jax.nn.sigmoid(gates[:, 3 * hh:4 * hh])
    c_new = gf * c[...] + gi * gg
    h_new = go * jnp.tanh(c_new)
    h_out[...] = h_new
    c_out[...] = c_new
    if nproj >= 1:
        rest[2][...] = jnp.dot(h_new, wp1_t[...],
                               preferred_element_type=F32) + bp1[...]
    if nproj >= 2:
        rest[3][...] = jnp.dot(h_new, wp2_t[...],
                               preferred_element_type=F32) + bp2[...]


def _lstm_call(y, aff_a, aff_b, h, c, wih_t, whh_t, bias, wp1_t, bp1, wp2_t,
               bp2, tile, nproj):
    n, fin = y.shape
    hh = h.shape[1]
    grid = n // tile
    full = lambda a: pl.BlockSpec(a.shape, lambda i: (0,) * a.ndim)
    in_specs = [
        pl.BlockSpec((tile, fin), lambda i: (i, 0)),
        full(aff_a), full(aff_b),
        pl.BlockSpec((tile, hh), lambda i: (i, 0)),
        pl.BlockSpec((tile, hh), lambda i: (i, 0)),
        full(wih_t), full(whh_t), full(bias), full(wp1_t), full(bp1),
        full(wp2_t), full(bp2),
    ]
    out_specs = [
        pl.BlockSpec((tile, hh), lambda i: (i, 0)),
        pl.BlockSpec((tile, hh), lambda i: (i, 0)),
    ]
    out_shape = [
        jax.ShapeDtypeStruct((n, hh), F32),
        jax.ShapeDtypeStruct((n, hh), F32),
    ]
    for k in range(nproj):
        wt = (wp1_t, wp2_t)[k]
        pw = wt.shape[1]
        out_specs.append(pl.BlockSpec((tile, pw), lambda i: (i, 0)))
        out_shape.append(jax.ShapeDtypeStruct((n, pw), F32))
    return pl.pallas_call(
        functools.partial(_lstm_body, nproj),
        grid=(grid,),
        in_specs=in_specs,
        out_specs=out_specs,
        out_shape=out_shape,
    )(y, aff_a, aff_b, h, c, wih_t, whh_t, bias, wp1_t, bp1, wp2_t, bp2)


def _edge_msg_body(y, aff_a, aff_b, h, c, g1, g3, wih_t, whh_t, bias, w2_t,
                   bmsg, h_out, c_out, ymsg_out, stats_out, acc):
    i = pl.program_id(0)
    xin = y[...] * aff_a[...] + aff_b[...]
    gates = (jnp.dot(xin, wih_t[...], preferred_element_type=F32)
             + jnp.dot(h[...], whh_t[...], preferred_element_type=F32)
             + bias[...])
    hh = h.shape[1]
    gi = jax.nn.sigmoid(gates[:, 0:hh])
    gf = jax.nn.sigmoid(gates[:, hh:2 * hh])
    gg = jnp.tanh(gates[:, 2 * hh:3 * hh])
    go = jax.nn.sigmoid(gates[:, 3 * hh:4 * hh])
    c_new = gf * c[...] + gi * gg
    h_new = go * jnp.tanh(c_new)
    h_out[...] = h_new
    c_out[...] = c_new
    z = (g1[...] + g3[...]
         + jnp.dot(h_new, w2_t[...], preferred_element_type=F32) + bmsg[...])
    ym = _leaky(z)
    ymsg_out[...] = ym

    @pl.when(i == 0)
    def _():
        acc[...] = jnp.zeros_like(acc)

    acc[0:1, :] = acc[0:1, :] + jnp.sum(ym, axis=0, keepdims=True)
    acc[1:2, :] = acc[1:2, :] + jnp.sum(ym * ym, axis=0, keepdims=True)
    stats_out[...] = acc[...]


def _edge_msg_call(y, aff_a, aff_b, h, c, g1, g3, wih_t, whh_t, bias, w2_t,
                   bmsg, tile):
    e, fin = y.shape
    hh = h.shape[1]
    fm = w2_t.shape[1]
    grid = e // tile
    full = lambda a: pl.BlockSpec(a.shape, lambda i: (0,) * a.ndim)
    row = lambda w: pl.BlockSpec((tile, w), lambda i: (i, 0))
    return pl.pallas_call(
        _edge_msg_body,
        grid=(grid,),
        in_specs=[row(fin), full(aff_a), full(aff_b), row(hh), row(hh),
                  row(fm), row(fm), full(wih_t), full(whh_t), full(bias),
                  full(w2_t), full(bmsg)],
        out_specs=[row(hh), row(hh), row(fm),
                   pl.BlockSpec((8, fm), lambda i: (0, 0))],
        out_shape=[jax.ShapeDtypeStruct((e, hh), F32),
                   jax.ShapeDtypeStruct((e, hh), F32),
                   jax.ShapeDtypeStruct((e, fm), F32),
                   jax.ShapeDtypeStruct((8, fm), F32)],
        scratch_shapes=[pltpu.VMEM((8, fm), F32)],
    )(y, aff_a, aff_b, h, c, g1, g3, wih_t, whh_t, bias, w2_t, bmsg)


def _node_msg_body(agg0, agg1, h, wa_t, wh_t, bias, y_out, stats_out, acc):
    i = pl.program_id(0)
    agg = agg0[...] + agg1[...]
    z = (jnp.dot(agg, wa_t[...], preferred_element_type=F32)
         + jnp.dot(h[...], wh_t[...], preferred_element_type=F32) + bias[...])
    ym = _leaky(z)
    y_out[...] = ym

    @pl.when(i == 0)
    def _():
        acc[...] = jnp.zeros_like(acc)

    acc[0:1, :] = acc[0:1, :] + jnp.sum(ym, axis=0, keepdims=True)
    acc[1:2, :] = acc[1:2, :] + jnp.sum(ym * ym, axis=0, keepdims=True)
    stats_out[...] = acc[...]


def _node_msg_call(agg2, h, wa_t, wh_t, bias, tile):
    n, hh = h.shape
    fo = wa_t.shape[1]
    grid = n // tile
    off = n // tile
    full = lambda a: pl.BlockSpec(a.shape, lambda i: (0,) * a.ndim)
    row = lambda w: pl.BlockSpec((tile, w), lambda i: (i, 0))
    return pl.pallas_call(
        _node_msg_body,
        grid=(grid,),
        in_specs=[row(hh),
                  pl.BlockSpec((tile, hh), lambda i: (i + off, 0)),
                  row(hh), full(wa_t), full(wh_t), full(bias)],
        out_specs=[row(fo), pl.BlockSpec((8, fo), lambda i: (0, 0))],
        out_shape=[jax.ShapeDtypeStruct((n, fo), F32),
                   jax.ShapeDtypeStruct((8, fo), F32)],
        scratch_shapes=[pltpu.VMEM((8, fo), F32)],
    )(agg2, agg2, h, wa_t, wh_t, bias)


_NWORK = 32
_CH = 100
_GRP = 5
_SUP = _CH * _GRP


def _gather_pair(p1, p3, src3, dst3):
    n, fm = p1.shape
    nch = src3.shape[1]
    e = _NWORK * nch * _CH
    ew = nch * _CH
    nsup = nch // _GRP
    assert nsup % 2 == 0
    mesh = plsc.VectorSubcoreMesh(core_axis_name="c", subcore_axis_name="s")

    @functools.partial(
        pl.kernel,
        out_type=(jax.ShapeDtypeStruct((e, fm), F32),
                  jax.ShapeDtypeStruct((e, fm), F32)),
        mesh=mesh,
        scratch_types=[
            pltpu.VMEM((nch, _CH), jnp.int32),
            pltpu.VMEM((nch, _CH), jnp.int32),
            pltpu.VMEM((2, _SUP, fm), F32),
            pltpu.VMEM((2, _SUP, fm), F32),
            pltpu.SemaphoreType.DMA,
            pltpu.SemaphoreType.DMA,
        ],
        compiler_params=pltpu.CompilerParams(use_tc_tiling_on_sc=False),
    )
    def gk(p1_hbm, p3_hbm, src_hbm, dst_hbm, g1_hbm, g3_hbm, idx_s, idx_d,
           r1, r2, semA, semB):
        c = lax.axis_index("c")
        s = lax.axis_index("s")
        wid = s * 2 + c
        pltpu.sync_copy(src_hbm.at[wid], idx_s)
        pltpu.sync_copy(dst_hbm.at[wid], idx_d)
        sems = (semA, semB)

        def fire(g, b):
            for k in range(_GRP):
                j = g * _GRP + k
                pltpu.async_copy(p1_hbm.at[idx_s.at[j]],
                                 r1.at[b].at[pl.ds(k * _CH, _CH)], sems[b])
                pltpu.async_copy(p3_hbm.at[idx_d.at[j]],
                                 r2.at[b].at[pl.ds(k * _CH, _CH)], sems[b])

        def drain_write(g, b):
            pltpu.make_async_copy(g1_hbm.at[pl.ds(0, _SUP)], r1.at[b],
                                  sems[b]).wait()
            pltpu.make_async_copy(g3_hbm.at[pl.ds(0, _SUP)], r2.at[b],
                                  sems[b]).wait()
            base = wid * ew + g * _SUP
            pltpu.sync_copy(r1.at[b], g1_hbm.at[pl.ds(base, _SUP)])
            pltpu.sync_copy(r2.at[b], g3_hbm.at[pl.ds(base, _SUP)])

        fire(0, 0)

        def body(u, carry):
            g = u * 2
            fire(g + 1, 1)
            drain_write(g, 0)

            @pl.when(u < nsup // 2 - 1)
            def _():
                fire(g + 2, 0)

            drain_write(g + 1, 1)
            return carry

        lax.fori_loop(0, nsup // 2, body, 0)

    return gk(p1, p3, src3, dst3)


def _segment_partials(h_e, dst3, zeros_nh):
    e, hh = h_e.shape
    n = zeros_nh.shape[0]
    nch = dst3.shape[1]
    ew = nch * _CH
    nsup = nch // _GRP
    assert nsup % 2 == 0
    rps = n // 16
    mesh = plsc.VectorSubcoreMesh(core_axis_name="c", subcore_axis_name="s")

    @functools.partial(
        pl.kernel,
        out_type=jax.ShapeDtypeStruct((2 * n, hh), F32),
        mesh=mesh,
        scratch_types=[
            pltpu.VMEM((nch, _CH), jnp.int32),
            pltpu.VMEM((2, _SUP, hh), F32),
            pltpu.VMEM_SHARED((n, hh), F32),
            pltpu.SemaphoreType.DMA,
            pltpu.SemaphoreType.DMA,
        ],
        compiler_params=pltpu.CompilerParams(use_tc_tiling_on_sc=False),
    )
    def sk(he_hbm, dst_hbm, zero_hbm, out_hbm, idx_d, buf, shared, semA,
           semB):
        c = lax.axis_index("c")
        s = lax.axis_index("s")
        wid = s * 2 + c
        pltpu.sync_copy(zero_hbm.at[pl.ds(s * rps, rps)],
                        shared.at[pl.ds(s * rps, rps)])
        pltpu.sync_copy(dst_hbm.at[wid], idx_d)
        plsc.subcore_barrier()
        sems = (semA, semB)

        def load(g, b):
            pltpu.async_copy(he_hbm.at[pl.ds(wid * ew + g * _SUP, _SUP)],
                             buf.at[b], sems[b])

        def scat(g, b):
            pltpu.make_async_copy(he_hbm.at[pl.ds(0, _SUP)], buf.at[b],
                                  sems[b]).wait()
            for k in range(_GRP):
                pltpu.sync_copy(buf.at[b].at[pl.ds(k * _CH, _CH)],
                                shared.at[idx_d.at[g * _GRP + k]], add=True)

        load(0, 0)

        def body(u, carry):
            g = u * 2
            load(g + 1, 1)
            scat(g, 0)

            @pl.when(u < nsup // 2 - 1)
            def _():
                load(g + 2, 0)

            scat(g + 1, 1)
            return carry

        lax.fori_loop(0, nsup // 2, body, 0)
        plsc.subcore_barrier()
        pltpu.sync_copy(shared.at[pl.ds(s * rps, rps)],
                        out_hbm.at[pl.ds(c * n + s * rps, rps)])

    return sk(h_e, dst3, zeros_nh)


def _bn_affine(stats, gamma, beta, count):
    s = stats[0, :]
    q = stats[1, :]
    m = s / count
    v = jnp.maximum(q / count - m * m, 0.0)
    a = gamma * lax.rsqrt(v + BN_EPS)
    b = beta - m * a
    return a[None, :], b[None, :]


def kernel(x, edge_feat, edge_index, Wih_n, Whh_n, bih_n, bhh_n, Wih_e,
           Whh_e, bih_e, bhh_e, W_nmpn, b_nmpn, W_empn, b_empn, gamma_n,
           beta_n, gamma_e, beta_e, W_fc, b_fc):
    n, nf = x.shape
    e, ef = edge_feat.shape
    hh = Whh_n.shape[1]
    nc = W_fc.shape[0]
    ew = e // _NWORK
    nch = ew // _CH

    wih_n_t = Wih_n.T
    whh_n_t = Whh_n.T
    bias_n = (bih_n + bhh_n)[None, :]
    wih_e_t = Wih_e.T
    whh_e_t = Whh_e.T
    bias_e = (bih_e + bhh_e)[None, :]
    w1_t = W_empn[:, 0:hh].T
    w2_t = W_empn[:, hh:2 * hh].T
    w3_t = W_empn[:, 2 * hh:3 * hh].T
    zp = jnp.zeros((1, ef), F32)
    wfc_t = W_fc.T
    bfc = b_fc[None, :]
    wa_t = W_nmpn[:, 0:hh].T
    wh_t = W_nmpn[:, hh:2 * hh].T
    bias_nm = b_nmpn[None, :]
    bmsg = b_empn[None, :]

    src3 = edge_index[0].reshape(_NWORK, nch, _CH)
    dst3 = edge_index[1].reshape(_NWORK, nch, _CH)
    zeros_nh = jnp.zeros((n, hh), F32)

    h_n = jnp.zeros((n, hh), F32)
    c_n = jnp.zeros((n, hh), F32)
    h_e = jnp.zeros((e, hh), F32)
    c_e = jnp.zeros((e, hh), F32)
    y_n = x
    y_e = edge_feat
    an = jnp.ones((1, nf), F32)
    bn = jnp.zeros((1, nf), F32)
    ae = jnp.ones((1, ef), F32)
    be = jnp.zeros((1, ef), F32)

    for it in range(NUM_ITER):
        last = it == NUM_ITER - 1
        if last:
            _, _, out = _lstm_call(y_n, an, bn, h_n, c_n, wih_n_t, whh_n_t,
                                   bias_n, wfc_t, bfc, w3_t, zp, tile=2000,
                                   nproj=1)
            return out
        if it <= NUM_ITER - 3:
            h_n, c_n, p1, p3 = _lstm_call(y_n, an, bn, h_n, c_n, wih_n_t,
                                          whh_n_t, bias_n, w1_t, zp, w3_t,
                                          zp, tile=2000, nproj=2)
            g1, g3 = _gather_pair(p1, p3, src3, dst3)
            h_e, c_e, y_e, stats_e = _edge_msg_call(
                y_e, ae, be, h_e, c_e, g1, g3, wih_e_t, whh_e_t, bias_e,
                w2_t, bmsg, tile=5000)
            ae, be = _bn_affine(stats_e, gamma_e, beta_e, float(e))
        else:
            h_n, c_n = _lstm_call(y_n, an, bn, h_n, c_n, wih_n_t, whh_n_t,
                                  bias_n, w1_t, zp, w3_t, zp, tile=2000,
                                  nproj=0)
            h_e, c_e = _lstm_call(y_e, ae, be, h_e, c_e, wih_e_t, whh_e_t,
                                  bias_e, w1_t, zp, w3_t, zp, tile=5000,
                                  nproj=0)
        agg2 = _segment_partials(h_e, dst3, zeros_nh)
        y_n, stats_n = _node_msg_call(agg2, h_n, wa_t, wh_t, bias_nm,
                                      tile=2000)
        an, bn = _bn_affine(stats_n, gamma_n, beta_n, float(n))
